# baseline (speedup 1.0000x reference)
import functools

import jax
import jax.numpy as jnp
from jax import lax
from jax.experimental import pallas as pl
from jax.experimental.pallas import tpu as pltpu
from jax.experimental.pallas import tpu_sc as plsc

N_CORES = 2
N_SUBCORES = 16
N_WORKERS = N_CORES * N_SUBCORES
BATCH = 64
NBUF = 4
IDX_CHUNK = 40
CNT_BATCH = 128
CNT_IDX_CHUNK = 16


def _fill(ref, nrows, d, value16):
    @pl.loop(0, nrows)
    def _(r):
        @pl.loop(0, d, step=16)
        def _(col):
            ref.at[pl.ds(r, 1), pl.ds(col, 16)][...] = value16


def _zero_acc_slice(zsrc, acc, base, rows_per_sub):
    chunk = zsrc.shape[0]
    nfull = rows_per_sub // chunk
    rem = rows_per_sub % chunk

    @pl.loop(0, nfull)
    def _(i):
        pltpu.sync_copy(zsrc, acc.at[pl.ds(base + i * chunk, chunk)])

    if rem:
        pltpu.sync_copy(zsrc.at[pl.ds(0, rem)],
                        acc.at[pl.ds(base + nfull * chunk, rem)])


def _make_seg_sum(n_pad, d, nb0, nb1):
    mesh = plsc.VectorSubcoreMesh(core_axis_name="c", subcore_axis_name="s")
    rows_per_sub = n_pad // N_SUBCORES
    scratch = (
        [
            pltpu.VMEM((IDX_CHUNK, BATCH), jnp.int32),
            pltpu.VMEM((IDX_CHUNK, BATCH), jnp.int32),
        ]
        + [pltpu.VMEM((BATCH, d), jnp.float32)] * NBUF
        + [pltpu.VMEM_SHARED((n_pad, d), jnp.float32)]
        + [pltpu.SemaphoreType.DMA] * NBUF
        + [pltpu.SemaphoreType.DMA] * NBUF
    )

    @functools.partial(
        pl.kernel,
        out_type=jax.ShapeDtypeStruct((N_CORES, n_pad, d), jnp.float32),
        mesh=mesh,
        scratch_types=scratch,
    )
    def seg_sum(y_hbm, src_hbm, dst_hbm, out_hbm, src_v, dst_v, *refs):
        rows = refs[:NBUF]
        acc = refs[NBUF]
        sem_g = refs[NBUF + 1:2 * NBUF + 1]
        sem_s = refs[2 * NBUF + 1:]
        c = lax.axis_index("c")
        s = lax.axis_index("s")
        blk0 = jnp.where(c == 0, 0, nb0)
        nchunks = jnp.where(c == 0, nb0 // IDX_CHUNK, nb1 // IDX_CHUNK)

        def gather(j, b):
            pltpu.async_copy(y_hbm.at[src_v.at[j]], rows[b], sem_g[b])

        def scatter(j, b):
            pltpu.async_copy(rows[b], acc.at[dst_v.at[j]], sem_s[b],
                             add=True)

        def wait_gather(b):
            pltpu.make_async_copy(y_hbm.at[src_v.at[0]], rows[b],
                                  sem_g[b]).wait()

        def wait_scatter(b):
            pltpu.make_async_copy(rows[b], acc.at[dst_v.at[0]],
                                  sem_s[b]).wait()

        _fill(rows[0], BATCH, d, jnp.zeros((1, 16), jnp.float32))
        base = s * rows_per_sub
        _zero_acc_slice(rows[0], acc, base, rows_per_sub)
        plsc.subcore_barrier()

        @pl.loop(0, nchunks)
        def _(ci):
            off = blk0 + ci * IDX_CHUNK
            pltpu.sync_copy(src_hbm.at[s, pl.ds(off, IDX_CHUNK)], src_v)
            pltpu.sync_copy(dst_hbm.at[s, pl.ds(off, IDX_CHUNK)], dst_v)
            gather(0, 0)
            gather(1, 1)

            @pl.loop(0, IDX_CHUNK // NBUF)
            def _(p):
                for b in range(NBUF):
                    j = p * NBUF + b
                    wait_gather(b)
                    scatter(j, b)
                    b2 = (b + 2) % NBUF

                    @pl.when(j >= 2)
                    def _():
                        wait_scatter(b2)

                    @pl.when(j + 2 < IDX_CHUNK)
                    def _():
                        gather(j + 2, b2)

            wait_scatter((IDX_CHUNK - 2) % NBUF)
            wait_scatter((IDX_CHUNK - 1) % NBUF)

        plsc.subcore_barrier()

        pltpu.sync_copy(acc.at[pl.ds(base, rows_per_sub)],
                        out_hbm.at[c, pl.ds(base, rows_per_sub)])

    return seg_sum


def _make_count(n_pad, nblocks):
    mesh = plsc.VectorSubcoreMesh(core_axis_name="c", subcore_axis_name="s")
    rows_per_sub = n_pad // N_SUBCORES
    nchunks = nblocks // CNT_IDX_CHUNK
    scratch = [
        pltpu.VMEM((CNT_IDX_CHUNK, CNT_BATCH), jnp.int32),
        pltpu.VMEM((CNT_BATCH, 128), jnp.float32),
        pltpu.VMEM_SHARED((n_pad, 128), jnp.float32),
        pltpu.SemaphoreType.DMA,
    ]

    @functools.partial(
        pl.kernel,
        out_type=jax.ShapeDtypeStruct((N_CORES, n_pad, 128), jnp.float32),
        mesh=mesh,
        scratch_types=scratch,
    )
    def count(dst_hbm, out_hbm, dst_v, ones_v, acc, sem):
        c = lax.axis_index("c")
        s = lax.axis_index("s")
        wid = s * N_CORES + c

        _fill(ones_v, CNT_BATCH, 128, jnp.zeros((1, 16), jnp.float32))
        base = s * rows_per_sub
        _zero_acc_slice(ones_v, acc, base, rows_per_sub)
        _fill(ones_v, CNT_BATCH, 128, jnp.ones((1, 16), jnp.float32))
        plsc.subcore_barrier()

        @pl.loop(0, nchunks)
        def _(ci):
            pltpu.sync_copy(
                dst_hbm.at[wid, pl.ds(ci * CNT_IDX_CHUNK, CNT_IDX_CHUNK)],
                dst_v)

            @pl.loop(0, CNT_IDX_CHUNK // 8)
            def _(q):
                @pl.loop(0, 8)
                def _(i):
                    pltpu.async_copy(ones_v, acc.at[dst_v.at[q * 8 + i]],
                                     sem, add=True)

                @pl.loop(0, 8)
                def _(i):
                    pltpu.make_async_copy(ones_v, acc.at[dst_v.at[0]],
                                          sem).wait()

        plsc.subcore_barrier()
        pltpu.sync_copy(acc.at[pl.ds(base, rows_per_sub)],
                        out_hbm.at[c, pl.ds(base, rows_per_sub)])

    return count


_ROW_BLK = 400


def _mm2_kernel(x_ref, wa_ref, wb_ref, a_ref, b_ref):
    xb = x_ref[...]
    a_ref[...] = jnp.dot(xb, wa_ref[...], preferred_element_type=jnp.float32)
    b_ref[...] = jnp.dot(xb, wb_ref[...], preferred_element_type=jnp.float32)


def _mm2(x, wa_t, wb_t):
    n, d = x.shape
    grid = n // _ROW_BLK
    w_spec = pl.BlockSpec((d, wa_t.shape[1]), lambda i: (0, 0))
    row_spec = pl.BlockSpec((_ROW_BLK, d), lambda i: (i, 0))
    out_spec = pl.BlockSpec((_ROW_BLK, wa_t.shape[1]), lambda i: (i, 0))
    return pl.pallas_call(
        _mm2_kernel,
        grid=(grid,),
        in_specs=[row_spec, w_spec, w_spec],
        out_specs=[out_spec, out_spec],
        out_shape=[
            jax.ShapeDtypeStruct((n, wa_t.shape[1]), jnp.float32),
            jax.ShapeDtypeStruct((n, wb_t.shape[1]), jnp.float32),
        ],
    )(x, wa_t, wb_t)


def _combine_mm2_kernel(aggp_ref, cntp_ref, r_ref, bl_ref, wa_ref, wb_ref,
                        a_ref, b_ref):
    agg = aggp_ref[0] + aggp_ref[1]
    cnt = cntp_ref[0, :, 0:1] + cntp_ref[1, :, 0:1]
    mean = agg / jnp.maximum(cnt, 1.0)
    h = jnp.maximum(mean + bl_ref[...] + r_ref[...], 0.0)
    a_ref[...] = jnp.dot(h, wa_ref[...], preferred_element_type=jnp.float32)
    b_ref[...] = jnp.dot(h, wb_ref[...], preferred_element_type=jnp.float32)


def _combine_mm2(aggp, cntp, r, bl, wa_t, wb_t):
    n, d = r.shape
    grid = n // _ROW_BLK
    return pl.pallas_call(
        _combine_mm2_kernel,
        grid=(grid,),
        in_specs=[
            pl.BlockSpec((N_CORES, _ROW_BLK, d), lambda i: (0, i, 0)),
            pl.BlockSpec((N_CORES, _ROW_BLK, 128), lambda i: (0, i, 0)),
            pl.BlockSpec((_ROW_BLK, d), lambda i: (i, 0)),
            pl.BlockSpec((d,), lambda i: (0,)),
            pl.BlockSpec((d, wa_t.shape[1]), lambda i: (0, 0)),
            pl.BlockSpec((d, wb_t.shape[1]), lambda i: (0, 0)),
        ],
        out_specs=[
            pl.BlockSpec((_ROW_BLK, wa_t.shape[1]), lambda i: (i, 0)),
            pl.BlockSpec((_ROW_BLK, wb_t.shape[1]), lambda i: (i, 0)),
        ],
        out_shape=[
            jax.ShapeDtypeStruct((n, wa_t.shape[1]), jnp.float32),
            jax.ShapeDtypeStruct((n, wb_t.shape[1]), jnp.float32),
        ],
    )(aggp, cntp, r, bl, wa_t, wb_t)


def _combine_out_kernel(aggp_ref, cntp_ref, r_ref, bl_ref, w_ref, blin_ref,
                        o_ref):
    agg = aggp_ref[0] + aggp_ref[1]
    cnt = cntp_ref[0, :, 0:1] + cntp_ref[1, :, 0:1]
    mean = agg / jnp.maximum(cnt, 1.0)
    h = jnp.maximum(mean + bl_ref[...] + r_ref[...], 0.0)
    o_ref[...] = (
        jnp.dot(h, w_ref[...], preferred_element_type=jnp.float32)
        + blin_ref[...]
    )


def _combine_out(aggp, cntp, r, bl, w_t, blin):
    n, d = r.shape
    grid = n // _ROW_BLK
    return pl.pallas_call(
        _combine_out_kernel,
        grid=(grid,),
        in_specs=[
            pl.BlockSpec((N_CORES, _ROW_BLK, d), lambda i: (0, i, 0)),
            pl.BlockSpec((N_CORES, _ROW_BLK, 128), lambda i: (0, i, 0)),
            pl.BlockSpec((_ROW_BLK, d), lambda i: (i, 0)),
            pl.BlockSpec((d,), lambda i: (0,)),
            pl.BlockSpec((d, w_t.shape[1]), lambda i: (0, 0)),
            pl.BlockSpec((w_t.shape[1],), lambda i: (0,)),
        ],
        out_specs=pl.BlockSpec((_ROW_BLK, w_t.shape[1]), lambda i: (i, 0)),
        out_shape=jax.ShapeDtypeStruct((n, w_t.shape[1]), jnp.float32),
    )(aggp, cntp, r, bl, w_t, blin)


def kernel(x, edge_index, Wl1, bl1, Wr1, Wl2, bl2, Wr2, Wlin, blin):
    n, d = x.shape
    e = edge_index.shape[1]

    nbt = -(-e // (N_SUBCORES * BATCH * IDX_CHUNK)) * IDX_CHUNK
    nb0 = min(nbt, max(IDX_CHUNK, round(1.0 * nbt / IDX_CHUNK) * IDX_CHUNK))
    nb1 = nbt - nb0
    e_pad = N_SUBCORES * nbt * BATCH
    cnt_nblocks = (-(-e // (N_WORKERS * CNT_BATCH * CNT_IDX_CHUNK))
                   * CNT_IDX_CHUNK)
    cnt_e_pad = N_WORKERS * cnt_nblocks * CNT_BATCH
    n_pad = -(-(n + 1) // (N_SUBCORES * 8)) * (N_SUBCORES * 8)

    pad = e_pad - e
    src = jnp.concatenate(
        [edge_index[0], jnp.zeros((pad,), jnp.int32)]
    ).reshape(N_SUBCORES, nbt, BATCH)
    dst = jnp.concatenate(
        [edge_index[1], jnp.full((pad,), n, jnp.int32)]
    ).reshape(N_SUBCORES, nbt, BATCH)
    dst_cnt = jnp.concatenate(
        [edge_index[1], jnp.full((cnt_e_pad - e,), n, jnp.int32)]
    ).reshape(N_WORKERS, cnt_nblocks, CNT_BATCH)

    seg_sum = _make_seg_sum(n_pad, d, nb0, nb1)
    count = _make_count(n_pad, cnt_nblocks)

    cntp = count(dst_cnt)
    y1, r1 = _mm2(x, Wl1.T, Wr1.T)
    aggp1 = seg_sum(y1, src, dst)
    y2, r2 = _combine_mm2(aggp1, cntp, r1, bl1, Wl2.T, Wr2.T)
    aggp2 = seg_sum(y2, src, dst)
    return _combine_out(aggp2, cntp, r2, bl2, Wlin.T, blin)


# speedup vs baseline: 1.2177x; 1.2177x over previous
import functools

import jax
import jax.numpy as jnp
from jax import lax
from jax.experimental import pallas as pl
from jax.experimental.pallas import tpu as pltpu
from jax.experimental.pallas import tpu_sc as plsc

N_CORES = 2
N_SUBCORES = 16
N_WORKERS = N_CORES * N_SUBCORES
BATCH = 64
NBUF = 4
IDX_CHUNK = 40
CNT_BATCH = 128
CNT_IDX_CHUNK = 16


def _fill(ref, nrows, d, value16):
    @pl.loop(0, nrows)
    def _(r):
        @pl.loop(0, d, step=16)
        def _(col):
            ref.at[pl.ds(r, 1), pl.ds(col, 16)][...] = value16


def _zero_acc_slice(zsrc, acc, base, rows_per_sub):
    chunk = zsrc.shape[0]
    nfull = rows_per_sub // chunk
    rem = rows_per_sub % chunk

    @pl.loop(0, nfull)
    def _(i):
        pltpu.sync_copy(zsrc, acc.at[pl.ds(base + i * chunk, chunk)])

    if rem:
        pltpu.sync_copy(zsrc.at[pl.ds(0, rem)],
                        acc.at[pl.ds(base + nfull * chunk, rem)])


def _make_seg_sum(n_pad, d, nb0, nb1):
    mesh = plsc.VectorSubcoreMesh(core_axis_name="c", subcore_axis_name="s")
    rows_per_sub = n_pad // N_SUBCORES
    scratch = (
        [
            pltpu.VMEM((IDX_CHUNK, BATCH), jnp.int32),
            pltpu.VMEM((IDX_CHUNK, BATCH), jnp.int32),
        ]
        + [pltpu.VMEM((BATCH, d), jnp.float32)] * NBUF
        + [pltpu.VMEM_SHARED((n_pad, d), jnp.float32)]
        + [pltpu.SemaphoreType.DMA] * NBUF
        + [pltpu.SemaphoreType.DMA] * NBUF
    )

    @functools.partial(
        pl.kernel,
        out_type=jax.ShapeDtypeStruct((N_CORES, n_pad, d), jnp.float32),
        mesh=mesh,
        scratch_types=scratch,
    )
    def seg_sum(y_hbm, src_hbm, dst_hbm, out_hbm, src_v, dst_v, *refs):
        rows = refs[:NBUF]
        acc = refs[NBUF]
        sem_g = refs[NBUF + 1:2 * NBUF + 1]
        sem_s = refs[2 * NBUF + 1:]
        c = lax.axis_index("c")
        s = lax.axis_index("s")
        blk0 = jnp.where(c == 0, 0, nb0)
        nchunks = jnp.where(c == 0, nb0 // IDX_CHUNK, nb1 // IDX_CHUNK)

        def gather(j, b):
            pltpu.async_copy(y_hbm.at[src_v.at[j]], rows[b], sem_g[b])

        def scatter(j, b):
            pltpu.async_copy(rows[b], acc.at[dst_v.at[j]], sem_s[b],
                             add=True)

        def wait_gather(b):
            pltpu.make_async_copy(y_hbm.at[src_v.at[0]], rows[b],
                                  sem_g[b]).wait()

        def wait_scatter(b):
            pltpu.make_async_copy(rows[b], acc.at[dst_v.at[0]],
                                  sem_s[b]).wait()

        _fill(rows[0], BATCH, d, jnp.zeros((1, 16), jnp.float32))
        base = s * rows_per_sub
        _zero_acc_slice(rows[0], acc, base, rows_per_sub)
        plsc.subcore_barrier()

        @pl.loop(0, nchunks)
        def _(ci):
            off = blk0 + ci * IDX_CHUNK
            pltpu.sync_copy(src_hbm.at[s, pl.ds(off, IDX_CHUNK)], src_v)
            pltpu.sync_copy(dst_hbm.at[s, pl.ds(off, IDX_CHUNK)], dst_v)
            gather(0, 0)
            gather(1, 1)

            @pl.loop(0, IDX_CHUNK // NBUF)
            def _(p):
                for b in range(NBUF):
                    j = p * NBUF + b
                    wait_gather(b)
                    scatter(j, b)
                    b2 = (b + 2) % NBUF

                    @pl.when(j >= 2)
                    def _():
                        wait_scatter(b2)

                    @pl.when(j + 2 < IDX_CHUNK)
                    def _():
                        gather(j + 2, b2)

            wait_scatter((IDX_CHUNK - 2) % NBUF)
            wait_scatter((IDX_CHUNK - 1) % NBUF)

        plsc.subcore_barrier()

        pltpu.sync_copy(acc.at[pl.ds(base, rows_per_sub)],
                        out_hbm.at[c, pl.ds(base, rows_per_sub)])

    return seg_sum


def _make_count(n_pad, nblocks):
    mesh = plsc.VectorSubcoreMesh(core_axis_name="c", subcore_axis_name="s")
    rows_per_sub = n_pad // N_SUBCORES
    nchunks = nblocks // CNT_IDX_CHUNK
    scratch = [
        pltpu.VMEM((CNT_IDX_CHUNK, CNT_BATCH), jnp.int32),
        pltpu.VMEM((CNT_BATCH, 128), jnp.float32),
        pltpu.VMEM_SHARED((n_pad, 128), jnp.float32),
        pltpu.SemaphoreType.DMA,
    ]

    @functools.partial(
        pl.kernel,
        out_type=jax.ShapeDtypeStruct((N_CORES, n_pad, 128), jnp.float32),
        mesh=mesh,
        scratch_types=scratch,
    )
    def count(dst_hbm, out_hbm, dst_v, ones_v, acc, sem):
        c = lax.axis_index("c")
        s = lax.axis_index("s")
        wid = s * N_CORES + c

        _fill(ones_v, CNT_BATCH, 128, jnp.zeros((1, 16), jnp.float32))
        base = s * rows_per_sub
        _zero_acc_slice(ones_v, acc, base, rows_per_sub)
        _fill(ones_v, CNT_BATCH, 128, jnp.ones((1, 16), jnp.float32))
        plsc.subcore_barrier()

        @pl.loop(0, nchunks)
        def _(ci):
            pltpu.sync_copy(
                dst_hbm.at[wid, pl.ds(ci * CNT_IDX_CHUNK, CNT_IDX_CHUNK)],
                dst_v)

            @pl.loop(0, CNT_IDX_CHUNK // 8)
            def _(q):
                @pl.loop(0, 8)
                def _(i):
                    pltpu.async_copy(ones_v, acc.at[dst_v.at[q * 8 + i]],
                                     sem, add=True)

                @pl.loop(0, 8)
                def _(i):
                    pltpu.make_async_copy(ones_v, acc.at[dst_v.at[0]],
                                          sem).wait()

        plsc.subcore_barrier()
        pltpu.sync_copy(acc.at[pl.ds(base, rows_per_sub)],
                        out_hbm.at[c, pl.ds(base, rows_per_sub)])

    return count


_ROW_BLK = 400


def _mm2_kernel(x_ref, wa_ref, wb_ref, a_ref, b_ref):
    xb = x_ref[...]
    a_ref[...] = jnp.dot(xb, wa_ref[...], preferred_element_type=jnp.float32)
    b_ref[...] = jnp.dot(xb, wb_ref[...], preferred_element_type=jnp.float32)


def _mm2(x, wa_t, wb_t):
    n, d = x.shape
    grid = n // _ROW_BLK
    w_spec = pl.BlockSpec((d, wa_t.shape[1]), lambda i: (0, 0))
    row_spec = pl.BlockSpec((_ROW_BLK, d), lambda i: (i, 0))
    out_spec = pl.BlockSpec((_ROW_BLK, wa_t.shape[1]), lambda i: (i, 0))
    return pl.pallas_call(
        _mm2_kernel,
        grid=(grid,),
        in_specs=[row_spec, w_spec, w_spec],
        out_specs=[out_spec, out_spec],
        out_shape=[
            jax.ShapeDtypeStruct((n, wa_t.shape[1]), jnp.float32),
            jax.ShapeDtypeStruct((n, wb_t.shape[1]), jnp.float32),
        ],
    )(x, wa_t, wb_t)


def _combine_mm2_kernel(aggp_ref, cntp_ref, r_ref, bl_ref, wa_ref, wb_ref,
                        a_ref, b_ref):
    agg = aggp_ref[0] + aggp_ref[1]
    cnt = cntp_ref[0, :, 0:1] + cntp_ref[1, :, 0:1]
    mean = agg / jnp.maximum(cnt, 1.0)
    h = jnp.maximum(mean + bl_ref[...] + r_ref[...], 0.0)
    a_ref[...] = jnp.dot(h, wa_ref[...], preferred_element_type=jnp.float32)
    b_ref[...] = jnp.dot(h, wb_ref[...], preferred_element_type=jnp.float32)


def _combine_mm2(aggp, cntp, r, bl, wa_t, wb_t):
    n, d = r.shape
    grid = n // _ROW_BLK
    return pl.pallas_call(
        _combine_mm2_kernel,
        grid=(grid,),
        in_specs=[
            pl.BlockSpec((N_CORES, _ROW_BLK, d), lambda i: (0, i, 0)),
            pl.BlockSpec((N_CORES, _ROW_BLK, 128), lambda i: (0, i, 0)),
            pl.BlockSpec((_ROW_BLK, d), lambda i: (i, 0)),
            pl.BlockSpec((d,), lambda i: (0,)),
            pl.BlockSpec((d, wa_t.shape[1]), lambda i: (0, 0)),
            pl.BlockSpec((d, wb_t.shape[1]), lambda i: (0, 0)),
        ],
        out_specs=[
            pl.BlockSpec((_ROW_BLK, wa_t.shape[1]), lambda i: (i, 0)),
            pl.BlockSpec((_ROW_BLK, wb_t.shape[1]), lambda i: (i, 0)),
        ],
        out_shape=[
            jax.ShapeDtypeStruct((n, wa_t.shape[1]), jnp.float32),
            jax.ShapeDtypeStruct((n, wb_t.shape[1]), jnp.float32),
        ],
    )(aggp, cntp, r, bl, wa_t, wb_t)


def _combine_out_kernel(aggp_ref, cntp_ref, r_ref, bl_ref, w_ref, blin_ref,
                        o_ref):
    agg = aggp_ref[0] + aggp_ref[1]
    cnt = cntp_ref[0, :, 0:1] + cntp_ref[1, :, 0:1]
    mean = agg / jnp.maximum(cnt, 1.0)
    h = jnp.maximum(mean + bl_ref[...] + r_ref[...], 0.0)
    o_ref[...] = (
        jnp.dot(h, w_ref[...], preferred_element_type=jnp.float32)
        + blin_ref[...]
    )


def _combine_out(aggp, cntp, r, bl, w_t, blin):
    n, d = r.shape
    grid = n // _ROW_BLK
    return pl.pallas_call(
        _combine_out_kernel,
        grid=(grid,),
        in_specs=[
            pl.BlockSpec((N_CORES, _ROW_BLK, d), lambda i: (0, i, 0)),
            pl.BlockSpec((N_CORES, _ROW_BLK, 128), lambda i: (0, i, 0)),
            pl.BlockSpec((_ROW_BLK, d), lambda i: (i, 0)),
            pl.BlockSpec((d,), lambda i: (0,)),
            pl.BlockSpec((d, w_t.shape[1]), lambda i: (0, 0)),
            pl.BlockSpec((w_t.shape[1],), lambda i: (0,)),
        ],
        out_specs=pl.BlockSpec((_ROW_BLK, w_t.shape[1]), lambda i: (i, 0)),
        out_shape=jax.ShapeDtypeStruct((n, w_t.shape[1]), jnp.float32),
    )(aggp, cntp, r, bl, w_t, blin)


def kernel(x, edge_index, Wl1, bl1, Wr1, Wl2, bl2, Wr2, Wlin, blin):
    n, d = x.shape
    e = edge_index.shape[1]

    nbt = -(-e // (N_SUBCORES * BATCH * IDX_CHUNK)) * IDX_CHUNK
    nb0 = min(nbt, max(IDX_CHUNK, round(0.625 * nbt / IDX_CHUNK) * IDX_CHUNK))
    nb1 = nbt - nb0
    e_pad = N_SUBCORES * nbt * BATCH
    cnt_nblocks = (-(-e // (N_WORKERS * CNT_BATCH * CNT_IDX_CHUNK))
                   * CNT_IDX_CHUNK)
    cnt_e_pad = N_WORKERS * cnt_nblocks * CNT_BATCH
    n_pad = -(-(n + 1) // (N_SUBCORES * 8)) * (N_SUBCORES * 8)

    pad = e_pad - e
    src = jnp.concatenate(
        [edge_index[0], jnp.zeros((pad,), jnp.int32)]
    ).reshape(N_SUBCORES, nbt, BATCH)
    dst = jnp.concatenate(
        [edge_index[1], jnp.full((pad,), n, jnp.int32)]
    ).reshape(N_SUBCORES, nbt, BATCH)
    dst_cnt = jnp.concatenate(
        [edge_index[1], jnp.full((cnt_e_pad - e,), n, jnp.int32)]
    ).reshape(N_WORKERS, cnt_nblocks, CNT_BATCH)

    seg_sum = _make_seg_sum(n_pad, d, nb0, nb1)
    count = _make_count(n_pad, cnt_nblocks)

    cntp = count(dst_cnt)
    y1, r1 = _mm2(x, Wl1.T, Wr1.T)
    aggp1 = seg_sum(y1, src, dst)
    y2, r2 = _combine_mm2(aggp1, cntp, r1, bl1, Wl2.T, Wr2.T)
    aggp2 = seg_sum(y2, src, dst)
    return _combine_out(aggp2, cntp, r2, bl2, Wlin.T, blin)


# speedup vs baseline: 1.5491x; 1.2721x over previous
import dataclasses
import functools

import jax
import jax.numpy as jnp
from jax import lax
from jax.experimental import pallas as pl
from jax.experimental.pallas import tpu as pltpu
from jax.experimental.pallas import tpu_sc as plsc

N_CORES = 2
N_SUBCORES = 16
N_WORKERS = N_CORES * N_SUBCORES
BATCH = 64
NBUF = 4
IDX_CHUNK = 16


def _fill(ref, nrows, d, value, flat):
    @pl.loop(0, nrows)
    def _(r):
        @pl.loop(0, d, step=16)
        def _(col):
            if flat:
                ref.at[r, pl.ds(col, 16)][...] = value
            else:
                ref.at[pl.ds(r, 1), pl.ds(col, 16)][...] = value


def _zero_acc_slice(zsrc, acc, base, rows_per_sub):
    chunk = zsrc.shape[0]
    nfull = rows_per_sub // chunk
    rem = rows_per_sub % chunk

    @pl.loop(0, nfull)
    def _(i):
        pltpu.sync_copy(zsrc, acc.at[pl.ds(base + i * chunk, chunk)])

    if rem:
        pltpu.sync_copy(zsrc.at[pl.ds(0, rem)],
                        acc.at[pl.ds(base + nfull * chunk, rem)])


def _make_seg_sum(n_pad, d, nb0, nb1, do_count=False):
    mesh = plsc.VectorSubcoreMesh(core_axis_name="c", subcore_axis_name="s")
    rows_per_sub = n_pad // N_SUBCORES
    scratch = (
        [
            pltpu.VMEM((IDX_CHUNK, BATCH), jnp.int32),
            pltpu.VMEM((IDX_CHUNK, BATCH), jnp.int32),
        ]
        + [pltpu.VMEM((BATCH, d), jnp.float32)] * NBUF
        + [pltpu.VMEM_SHARED((n_pad, d), jnp.float32)]
        + [pltpu.SemaphoreType.DMA] * NBUF
        + [pltpu.SemaphoreType.DMA] * NBUF
    )
    out_type = jax.ShapeDtypeStruct((N_CORES, n_pad, d), jnp.float32)
    cp = None
    if do_count:
        out_type = [out_type,
                    jax.ShapeDtypeStruct((N_WORKERS, n_pad), jnp.float32)]
        scratch = scratch + [pltpu.VMEM((n_pad,), jnp.float32)]
        cp = pltpu.CompilerParams()
        if "needs_layout_passes" in pltpu.CompilerParams.__dataclass_fields__:
            cp = dataclasses.replace(cp, needs_layout_passes=False)

    @functools.partial(
        pl.kernel,
        out_type=out_type,
        mesh=mesh,
        scratch_types=scratch,
        compiler_params=cp,
    )
    def seg_sum(y_hbm, src_hbm, dst_hbm, *args):
        if do_count:
            out_hbm, cnt_hbm, src_v, dst_v, *refs = args
            cnt_v = refs[-1]
            refs = refs[:-1]
        else:
            out_hbm, src_v, dst_v, *refs = args
        rows = refs[:NBUF]
        acc = refs[NBUF]
        sem_g = refs[NBUF + 1:2 * NBUF + 1]
        sem_s = refs[2 * NBUF + 1:]
        c = lax.axis_index("c")
        s = lax.axis_index("s")
        blk0 = jnp.where(c == 0, 0, nb0)
        nchunks = jnp.where(c == 0, nb0 // IDX_CHUNK, nb1 // IDX_CHUNK)

        def gather(j, b):
            pltpu.async_copy(y_hbm.at[src_v.at[j]], rows[b], sem_g[b])

        def scatter(j, b):
            pltpu.async_copy(rows[b], acc.at[dst_v.at[j]], sem_s[b],
                             add=True)

        def wait_gather(b):
            pltpu.make_async_copy(y_hbm.at[src_v.at[0]], rows[b],
                                  sem_g[b]).wait()

        def wait_scatter(b):
            pltpu.make_async_copy(rows[b], acc.at[dst_v.at[0]],
                                  sem_s[b]).wait()

        zshape = (16,) if do_count else (1, 16)
        _fill(rows[0], BATCH, d, jnp.zeros(zshape, jnp.float32), do_count)
        base = s * rows_per_sub
        _zero_acc_slice(rows[0], acc, base, rows_per_sub)
        if do_count:
            z16 = jnp.zeros((16,), jnp.float32)

            @pl.loop(0, n_pad, step=16)
            def _(i):
                cnt_v.at[pl.ds(i, 16)][...] = z16
        one16 = jnp.ones((16,), jnp.float32)
        plsc.subcore_barrier()

        @pl.loop(0, nchunks)
        def _(ci):
            off = blk0 + ci * IDX_CHUNK
            pltpu.sync_copy(src_hbm.at[s, pl.ds(off, IDX_CHUNK)], src_v)
            pltpu.sync_copy(dst_hbm.at[s, pl.ds(off, IDX_CHUNK)], dst_v)
            gather(0, 0)
            gather(1, 1)

            @pl.loop(0, IDX_CHUNK // NBUF)
            def _(p):
                for b in range(NBUF):
                    j = p * NBUF + b
                    wait_gather(b)
                    scatter(j, b)
                    if do_count:
                        @pl.loop(0, BATCH, step=16)
                        def _(k):
                            plsc.addupdate_scatter(
                                cnt_v, [dst_v.at[j, pl.ds(k, 16)][...]],
                                one16)
                    b2 = (b + 2) % NBUF

                    @pl.when(j >= 2)
                    def _():
                        wait_scatter(b2)

                    @pl.when(j + 2 < IDX_CHUNK)
                    def _():
                        gather(j + 2, b2)

            wait_scatter((IDX_CHUNK - 2) % NBUF)
            wait_scatter((IDX_CHUNK - 1) % NBUF)

        if do_count:
            pltpu.sync_copy(cnt_v, cnt_hbm.at[s * N_CORES + c])
        plsc.subcore_barrier()

        pltpu.sync_copy(acc.at[pl.ds(base, rows_per_sub)],
                        out_hbm.at[c, pl.ds(base, rows_per_sub)])

    return seg_sum


_ROW_BLK = 400


def _mm2_kernel(x_ref, wa_ref, wb_ref, a_ref, b_ref):
    xb = x_ref[...]
    a_ref[...] = jnp.dot(xb, wa_ref[...], preferred_element_type=jnp.float32)
    b_ref[...] = jnp.dot(xb, wb_ref[...], preferred_element_type=jnp.float32)


def _mm2(x, wa_t, wb_t):
    n, d = x.shape
    grid = n // _ROW_BLK
    w_spec = pl.BlockSpec((d, wa_t.shape[1]), lambda i: (0, 0))
    row_spec = pl.BlockSpec((_ROW_BLK, d), lambda i: (i, 0))
    out_spec = pl.BlockSpec((_ROW_BLK, wa_t.shape[1]), lambda i: (i, 0))
    return pl.pallas_call(
        _mm2_kernel,
        grid=(grid,),
        in_specs=[row_spec, w_spec, w_spec],
        out_specs=[out_spec, out_spec],
        out_shape=[
            jax.ShapeDtypeStruct((n, wa_t.shape[1]), jnp.float32),
            jax.ShapeDtypeStruct((n, wb_t.shape[1]), jnp.float32),
        ],
    )(x, wa_t, wb_t)


def _combine_mm2_kernel(aggp_ref, cntp_ref, r_ref, bl_ref, wa_ref, wb_ref,
                        a_ref, b_ref):
    agg = aggp_ref[0] + aggp_ref[1]
    cnt = jnp.sum(cntp_ref[...], axis=1)[:, None]
    mean = agg / jnp.maximum(cnt, 1.0)
    h = jnp.maximum(mean + bl_ref[...] + r_ref[...], 0.0)
    a_ref[...] = jnp.dot(h, wa_ref[...], preferred_element_type=jnp.float32)
    b_ref[...] = jnp.dot(h, wb_ref[...], preferred_element_type=jnp.float32)


def _combine_mm2(aggp, cntp, r, bl, wa_t, wb_t):
    n, d = r.shape
    grid = n // _ROW_BLK
    return pl.pallas_call(
        _combine_mm2_kernel,
        grid=(grid,),
        in_specs=[
            pl.BlockSpec((N_CORES, _ROW_BLK, d), lambda i: (0, i, 0)),
            pl.BlockSpec((_ROW_BLK, N_WORKERS), lambda i: (i, 0)),
            pl.BlockSpec((_ROW_BLK, d), lambda i: (i, 0)),
            pl.BlockSpec((d,), lambda i: (0,)),
            pl.BlockSpec((d, wa_t.shape[1]), lambda i: (0, 0)),
            pl.BlockSpec((d, wb_t.shape[1]), lambda i: (0, 0)),
        ],
        out_specs=[
            pl.BlockSpec((_ROW_BLK, wa_t.shape[1]), lambda i: (i, 0)),
            pl.BlockSpec((_ROW_BLK, wb_t.shape[1]), lambda i: (i, 0)),
        ],
        out_shape=[
            jax.ShapeDtypeStruct((n, wa_t.shape[1]), jnp.float32),
            jax.ShapeDtypeStruct((n, wb_t.shape[1]), jnp.float32),
        ],
    )(aggp, cntp, r, bl, wa_t, wb_t)


def _combine_out_kernel(aggp_ref, cntp_ref, r_ref, bl_ref, w_ref, blin_ref,
                        o_ref):
    agg = aggp_ref[0] + aggp_ref[1]
    cnt = jnp.sum(cntp_ref[...], axis=1)[:, None]
    mean = agg / jnp.maximum(cnt, 1.0)
    h = jnp.maximum(mean + bl_ref[...] + r_ref[...], 0.0)
    o_ref[...] = (
        jnp.dot(h, w_ref[...], preferred_element_type=jnp.float32)
        + blin_ref[...]
    )


def _combine_out(aggp, cntp, r, bl, w_t, blin):
    n, d = r.shape
    grid = n // _ROW_BLK
    return pl.pallas_call(
        _combine_out_kernel,
        grid=(grid,),
        in_specs=[
            pl.BlockSpec((N_CORES, _ROW_BLK, d), lambda i: (0, i, 0)),
            pl.BlockSpec((_ROW_BLK, N_WORKERS), lambda i: (i, 0)),
            pl.BlockSpec((_ROW_BLK, d), lambda i: (i, 0)),
            pl.BlockSpec((d,), lambda i: (0,)),
            pl.BlockSpec((d, w_t.shape[1]), lambda i: (0, 0)),
            pl.BlockSpec((w_t.shape[1],), lambda i: (0,)),
        ],
        out_specs=pl.BlockSpec((_ROW_BLK, w_t.shape[1]), lambda i: (i, 0)),
        out_shape=jax.ShapeDtypeStruct((n, w_t.shape[1]), jnp.float32),
    )(aggp, cntp, r, bl, w_t, blin)


def kernel(x, edge_index, Wl1, bl1, Wr1, Wl2, bl2, Wr2, Wlin, blin):
    n, d = x.shape
    e = edge_index.shape[1]

    nbt = -(-e // (N_SUBCORES * BATCH * IDX_CHUNK)) * IDX_CHUNK
    nb0 = min(nbt, max(IDX_CHUNK, round(0.75 * nbt / IDX_CHUNK) * IDX_CHUNK))
    nb1 = nbt - nb0
    e_pad = N_SUBCORES * nbt * BATCH
    n_pad = -(-(n + 1) // (N_SUBCORES * 8)) * (N_SUBCORES * 8)

    pad = e_pad - e
    src = jnp.concatenate(
        [edge_index[0], jnp.zeros((pad,), jnp.int32)]
    ).reshape(N_SUBCORES, nbt, BATCH)
    dst = jnp.concatenate(
        [edge_index[1], jnp.full((pad,), n, jnp.int32)]
    ).reshape(N_SUBCORES, nbt, BATCH)
    seg_sum_cnt = _make_seg_sum(n_pad, d, nb0, nb1, do_count=True)
    seg_sum = _make_seg_sum(n_pad, d, nb0, nb1)

    y1, r1 = _mm2(x, Wl1.T, Wr1.T)
    aggp1, cntp = seg_sum_cnt(y1, src, dst)
    cntp = cntp.T
    y2, r2 = _combine_mm2(aggp1, cntp, r1, bl1, Wl2.T, Wr2.T)
    aggp2 = seg_sum(y2, src, dst)
    return _combine_out(aggp2, cntp, r2, bl2, Wlin.T, blin)


# speedup vs baseline: 1.5522x; 1.0020x over previous
import dataclasses
import functools

import jax
import jax.numpy as jnp
from jax import lax
from jax.experimental import pallas as pl
from jax.experimental.pallas import tpu as pltpu
from jax.experimental.pallas import tpu_sc as plsc

N_CORES = 2
N_SUBCORES = 16
N_WORKERS = N_CORES * N_SUBCORES
BATCH = 64
NBUF = 4
IDX_CHUNK = 16


def _fill(ref, nrows, d, value, flat):
    @pl.loop(0, nrows)
    def _(r):
        @pl.loop(0, d, step=16)
        def _(col):
            if flat:
                ref.at[r, pl.ds(col, 16)][...] = value
            else:
                ref.at[pl.ds(r, 1), pl.ds(col, 16)][...] = value


def _zero_acc_slice(zsrc, acc, base, rows_per_sub):
    chunk = zsrc.shape[0]
    nfull = rows_per_sub // chunk
    rem = rows_per_sub % chunk

    @pl.loop(0, nfull)
    def _(i):
        pltpu.sync_copy(zsrc, acc.at[pl.ds(base + i * chunk, chunk)])

    if rem:
        pltpu.sync_copy(zsrc.at[pl.ds(0, rem)],
                        acc.at[pl.ds(base + nfull * chunk, rem)])


def _make_seg_sum(n_pad, d, nb0, nb1, do_count=False, ich=IDX_CHUNK):
    mesh = plsc.VectorSubcoreMesh(core_axis_name="c", subcore_axis_name="s")
    rows_per_sub = n_pad // N_SUBCORES
    scratch = (
        [
            pltpu.VMEM((ich, BATCH), jnp.int32),
            pltpu.VMEM((ich, BATCH), jnp.int32),
        ]
        + [pltpu.VMEM((BATCH, d), jnp.float32)] * NBUF
        + [pltpu.VMEM_SHARED((n_pad, d), jnp.float32)]
        + [pltpu.SemaphoreType.DMA] * NBUF
        + [pltpu.SemaphoreType.DMA] * NBUF
    )
    out_type = jax.ShapeDtypeStruct((N_CORES, n_pad, d), jnp.float32)
    cp = None
    if do_count:
        out_type = [out_type,
                    jax.ShapeDtypeStruct((N_WORKERS, n_pad), jnp.float32)]
        scratch = scratch + [pltpu.VMEM((n_pad,), jnp.float32)]
        cp = pltpu.CompilerParams()
        if "needs_layout_passes" in pltpu.CompilerParams.__dataclass_fields__:
            cp = dataclasses.replace(cp, needs_layout_passes=False)

    @functools.partial(
        pl.kernel,
        out_type=out_type,
        mesh=mesh,
        scratch_types=scratch,
        compiler_params=cp,
    )
    def seg_sum(y_hbm, src_hbm, dst_hbm, *args):
        if do_count:
            out_hbm, cnt_hbm, src_v, dst_v, *refs = args
            cnt_v = refs[-1]
            refs = refs[:-1]
        else:
            out_hbm, src_v, dst_v, *refs = args
        rows = refs[:NBUF]
        acc = refs[NBUF]
        sem_g = refs[NBUF + 1:2 * NBUF + 1]
        sem_s = refs[2 * NBUF + 1:]
        c = lax.axis_index("c")
        s = lax.axis_index("s")
        blk0 = jnp.where(c == 0, 0, nb0)
        nchunks = jnp.where(c == 0, nb0 // ich, nb1 // ich)

        def gather(j, b):
            pltpu.async_copy(y_hbm.at[src_v.at[j]], rows[b], sem_g[b])

        def scatter(j, b):
            pltpu.async_copy(rows[b], acc.at[dst_v.at[j]], sem_s[b],
                             add=True)

        def wait_gather(b):
            pltpu.make_async_copy(y_hbm.at[src_v.at[0]], rows[b],
                                  sem_g[b]).wait()

        def wait_scatter(b):
            pltpu.make_async_copy(rows[b], acc.at[dst_v.at[0]],
                                  sem_s[b]).wait()

        zshape = (16,) if do_count else (1, 16)
        _fill(rows[0], BATCH, d, jnp.zeros(zshape, jnp.float32), do_count)
        base = s * rows_per_sub
        _zero_acc_slice(rows[0], acc, base, rows_per_sub)
        if do_count:
            z16 = jnp.zeros((16,), jnp.float32)

            @pl.loop(0, n_pad, step=16)
            def _(i):
                cnt_v.at[pl.ds(i, 16)][...] = z16
        one16 = jnp.ones((16,), jnp.float32)
        plsc.subcore_barrier()

        @pl.loop(0, nchunks)
        def _(ci):
            off = blk0 + ci * ich
            pltpu.sync_copy(src_hbm.at[s, pl.ds(off, ich)], src_v)
            pltpu.sync_copy(dst_hbm.at[s, pl.ds(off, ich)], dst_v)
            gather(0, 0)
            gather(1, 1)

            @pl.loop(0, ich // NBUF)
            def _(p):
                for b in range(NBUF):
                    j = p * NBUF + b
                    wait_gather(b)
                    scatter(j, b)
                    if do_count:
                        @pl.loop(0, BATCH, step=16)
                        def _(k):
                            plsc.addupdate_scatter(
                                cnt_v, [dst_v.at[j, pl.ds(k, 16)][...]],
                                one16)
                    b2 = (b + 2) % NBUF

                    @pl.when(j >= 2)
                    def _():
                        wait_scatter(b2)

                    @pl.when(j + 2 < ich)
                    def _():
                        gather(j + 2, b2)

            wait_scatter((ich - 2) % NBUF)
            wait_scatter((ich - 1) % NBUF)

        if do_count:
            pltpu.sync_copy(cnt_v, cnt_hbm.at[s * N_CORES + c])
        plsc.subcore_barrier()

        pltpu.sync_copy(acc.at[pl.ds(base, rows_per_sub)],
                        out_hbm.at[c, pl.ds(base, rows_per_sub)])

    return seg_sum


_ROW_BLK = 400


def _mm2_kernel(x_ref, wa_ref, wb_ref, a_ref, b_ref):
    xb = x_ref[...]
    a_ref[...] = jnp.dot(xb, wa_ref[...], preferred_element_type=jnp.float32)
    b_ref[...] = jnp.dot(xb, wb_ref[...], preferred_element_type=jnp.float32)


def _mm2(x, wa_t, wb_t):
    n, d = x.shape
    grid = n // _ROW_BLK
    w_spec = pl.BlockSpec((d, wa_t.shape[1]), lambda i: (0, 0))
    row_spec = pl.BlockSpec((_ROW_BLK, d), lambda i: (i, 0))
    out_spec = pl.BlockSpec((_ROW_BLK, wa_t.shape[1]), lambda i: (i, 0))
    return pl.pallas_call(
        _mm2_kernel,
        grid=(grid,),
        in_specs=[row_spec, w_spec, w_spec],
        out_specs=[out_spec, out_spec],
        out_shape=[
            jax.ShapeDtypeStruct((n, wa_t.shape[1]), jnp.float32),
            jax.ShapeDtypeStruct((n, wb_t.shape[1]), jnp.float32),
        ],
    )(x, wa_t, wb_t)


def _combine_mm2_kernel(aggp_ref, cntp_ref, r_ref, bl_ref, wa_ref, wb_ref,
                        a_ref, b_ref):
    agg = aggp_ref[0] + aggp_ref[1]
    cnt = jnp.sum(cntp_ref[...], axis=1)[:, None]
    mean = agg / jnp.maximum(cnt, 1.0)
    h = jnp.maximum(mean + bl_ref[...] + r_ref[...], 0.0)
    a_ref[...] = jnp.dot(h, wa_ref[...], preferred_element_type=jnp.float32)
    b_ref[...] = jnp.dot(h, wb_ref[...], preferred_element_type=jnp.float32)


def _combine_mm2(aggp, cntp, r, bl, wa_t, wb_t):
    n, d = r.shape
    grid = n // _ROW_BLK
    return pl.pallas_call(
        _combine_mm2_kernel,
        grid=(grid,),
        in_specs=[
            pl.BlockSpec((N_CORES, _ROW_BLK, d), lambda i: (0, i, 0)),
            pl.BlockSpec((_ROW_BLK, N_WORKERS), lambda i: (i, 0)),
            pl.BlockSpec((_ROW_BLK, d), lambda i: (i, 0)),
            pl.BlockSpec((d,), lambda i: (0,)),
            pl.BlockSpec((d, wa_t.shape[1]), lambda i: (0, 0)),
            pl.BlockSpec((d, wb_t.shape[1]), lambda i: (0, 0)),
        ],
        out_specs=[
            pl.BlockSpec((_ROW_BLK, wa_t.shape[1]), lambda i: (i, 0)),
            pl.BlockSpec((_ROW_BLK, wb_t.shape[1]), lambda i: (i, 0)),
        ],
        out_shape=[
            jax.ShapeDtypeStruct((n, wa_t.shape[1]), jnp.float32),
            jax.ShapeDtypeStruct((n, wb_t.shape[1]), jnp.float32),
        ],
    )(aggp, cntp, r, bl, wa_t, wb_t)


def _combine_out_kernel(aggp_ref, cntp_ref, r_ref, bl_ref, w_ref, blin_ref,
                        o_ref):
    agg = aggp_ref[0] + aggp_ref[1]
    cnt = jnp.sum(cntp_ref[...], axis=1)[:, None]
    mean = agg / jnp.maximum(cnt, 1.0)
    h = jnp.maximum(mean + bl_ref[...] + r_ref[...], 0.0)
    o_ref[...] = (
        jnp.dot(h, w_ref[...], preferred_element_type=jnp.float32)
        + blin_ref[...]
    )


def _combine_out(aggp, cntp, r, bl, w_t, blin):
    n, d = r.shape
    grid = n // _ROW_BLK
    return pl.pallas_call(
        _combine_out_kernel,
        grid=(grid,),
        in_specs=[
            pl.BlockSpec((N_CORES, _ROW_BLK, d), lambda i: (0, i, 0)),
            pl.BlockSpec((_ROW_BLK, N_WORKERS), lambda i: (i, 0)),
            pl.BlockSpec((_ROW_BLK, d), lambda i: (i, 0)),
            pl.BlockSpec((d,), lambda i: (0,)),
            pl.BlockSpec((d, w_t.shape[1]), lambda i: (0, 0)),
            pl.BlockSpec((w_t.shape[1],), lambda i: (0,)),
        ],
        out_specs=pl.BlockSpec((_ROW_BLK, w_t.shape[1]), lambda i: (i, 0)),
        out_shape=jax.ShapeDtypeStruct((n, w_t.shape[1]), jnp.float32),
    )(aggp, cntp, r, bl, w_t, blin)


def kernel(x, edge_index, Wl1, bl1, Wr1, Wl2, bl2, Wr2, Wlin, blin):
    n, d = x.shape
    e = edge_index.shape[1]

    nbt = -(-e // (N_SUBCORES * BATCH * IDX_CHUNK)) * IDX_CHUNK
    nb0 = min(nbt, max(IDX_CHUNK, round(0.75 * nbt / IDX_CHUNK) * IDX_CHUNK))
    nb1 = nbt - nb0
    e_pad = N_SUBCORES * nbt * BATCH
    n_pad = -(-(n + 1) // (N_SUBCORES * 8)) * (N_SUBCORES * 8)

    pad = e_pad - e
    src = jnp.concatenate(
        [edge_index[0], jnp.zeros((pad,), jnp.int32)]
    ).reshape(N_SUBCORES, nbt, BATCH)
    dst = jnp.concatenate(
        [edge_index[1], jnp.full((pad,), n, jnp.int32)]
    ).reshape(N_SUBCORES, nbt, BATCH)
    seg_sum_cnt = _make_seg_sum(n_pad, d, nb0, nb1, do_count=True)
    ich2 = 40 if (nb0 % 40 == 0 and nb1 % 40 == 0) else IDX_CHUNK
    seg_sum = _make_seg_sum(n_pad, d, nb0, nb1, ich=ich2)

    y1, r1 = _mm2(x, Wl1.T, Wr1.T)
    aggp1, cntp = seg_sum_cnt(y1, src, dst)
    cntp = cntp.T
    y2, r2 = _combine_mm2(aggp1, cntp, r1, bl1, Wl2.T, Wr2.T)
    aggp2 = seg_sum(y2, src, dst)
    return _combine_out(aggp2, cntp, r2, bl2, Wlin.T, blin)


# speedup vs baseline: 1.6193x; 1.0433x over previous
import dataclasses
import functools

import jax
import jax.numpy as jnp
from jax import lax
from jax.experimental import pallas as pl
from jax.experimental.pallas import tpu as pltpu
from jax.experimental.pallas import tpu_sc as plsc

N_CORES = 2
N_SUBCORES = 16
N_WORKERS = N_CORES * N_SUBCORES
BATCH = 64
NBUF = 4
IDX_CHUNK = 16


def _fill(ref, nrows, d, value, flat):
    @pl.loop(0, nrows)
    def _(r):
        @pl.loop(0, d, step=16)
        def _(col):
            if flat:
                ref.at[r, pl.ds(col, 16)][...] = value
            else:
                ref.at[pl.ds(r, 1), pl.ds(col, 16)][...] = value


def _zero_acc_slice(zsrc, acc, base, rows_per_sub):
    chunk = zsrc.shape[0]
    nfull = rows_per_sub // chunk
    rem = rows_per_sub % chunk

    @pl.loop(0, nfull)
    def _(i):
        pltpu.sync_copy(zsrc, acc.at[pl.ds(base + i * chunk, chunk)])

    if rem:
        pltpu.sync_copy(zsrc.at[pl.ds(0, rem)],
                        acc.at[pl.ds(base + nfull * chunk, rem)])


def _make_seg_sum(n_pad, d, nb0, nb1, do_count=False, ich=IDX_CHUNK):
    mesh = plsc.VectorSubcoreMesh(core_axis_name="c", subcore_axis_name="s")
    rows_per_sub = n_pad // N_SUBCORES
    scratch = (
        [
            pltpu.VMEM((ich, BATCH), jnp.int32),
            pltpu.VMEM((ich, BATCH), jnp.int32),
        ]
        + [pltpu.VMEM((BATCH, d), jnp.float32)] * NBUF
        + [pltpu.VMEM_SHARED((n_pad, d), jnp.float32)]
        + [pltpu.SemaphoreType.DMA] * NBUF
        + [pltpu.SemaphoreType.DMA] * NBUF
    )
    out_type = jax.ShapeDtypeStruct((N_CORES, n_pad, d), jnp.float32)
    cp = None
    if do_count:
        out_type = [out_type,
                    jax.ShapeDtypeStruct((N_WORKERS, n_pad), jnp.float32)]
        scratch = scratch + [pltpu.VMEM((n_pad,), jnp.float32)]
        cp = pltpu.CompilerParams()
        if "needs_layout_passes" in pltpu.CompilerParams.__dataclass_fields__:
            cp = dataclasses.replace(cp, needs_layout_passes=False)

    @functools.partial(
        pl.kernel,
        out_type=out_type,
        mesh=mesh,
        scratch_types=scratch,
        compiler_params=cp,
    )
    def seg_sum(y_hbm, src_hbm, dst_hbm, *args):
        if do_count:
            out_hbm, cnt_hbm, src_v, dst_v, *refs = args
            cnt_v = refs[-1]
            refs = refs[:-1]
        else:
            out_hbm, src_v, dst_v, *refs = args
        rows = refs[:NBUF]
        acc = refs[NBUF]
        sem_g = refs[NBUF + 1:2 * NBUF + 1]
        sem_s = refs[2 * NBUF + 1:]
        c = lax.axis_index("c")
        s = lax.axis_index("s")
        blk0 = jnp.where(c == 0, 0, nb0)
        nchunks = jnp.where(c == 0, nb0 // ich, nb1 // ich)

        def gather(j, b):
            pltpu.async_copy(y_hbm.at[src_v.at[j]], rows[b], sem_g[b])

        def scatter(j, b):
            pltpu.async_copy(rows[b], acc.at[dst_v.at[j]], sem_s[b],
                             add=True)

        def wait_gather(b):
            pltpu.make_async_copy(y_hbm.at[src_v.at[0]], rows[b],
                                  sem_g[b]).wait()

        def wait_scatter(b):
            pltpu.make_async_copy(rows[b], acc.at[dst_v.at[0]],
                                  sem_s[b]).wait()

        zshape = (16,) if do_count else (1, 16)
        _fill(rows[0], BATCH, d, jnp.zeros(zshape, jnp.float32), do_count)
        base = s * rows_per_sub
        _zero_acc_slice(rows[0], acc, base, rows_per_sub)
        if do_count:
            z16 = jnp.zeros((16,), jnp.float32)

            @pl.loop(0, n_pad, step=16)
            def _(i):
                cnt_v.at[pl.ds(i, 16)][...] = z16
        one16 = jnp.ones((16,), jnp.float32)
        plsc.subcore_barrier()

        @pl.loop(0, nchunks)
        def _(ci):
            off = blk0 + ci * ich
            pltpu.sync_copy(src_hbm.at[s, pl.ds(off, ich)], src_v)
            pltpu.sync_copy(dst_hbm.at[s, pl.ds(off, ich)], dst_v)
            gather(0, 0)
            gather(1, 1)

            @pl.loop(0, ich // NBUF)
            def _(p):
                for b in range(NBUF):
                    j = p * NBUF + b
                    wait_gather(b)
                    scatter(j, b)
                    if do_count:
                        @pl.loop(0, BATCH, step=16)
                        def _(k):
                            plsc.addupdate_scatter(
                                cnt_v, [dst_v.at[j, pl.ds(k, 16)][...]],
                                one16)
                    b2 = (b + 2) % NBUF

                    @pl.when(j >= 2)
                    def _():
                        wait_scatter(b2)

                    @pl.when(j + 2 < ich)
                    def _():
                        gather(j + 2, b2)

            wait_scatter((ich - 2) % NBUF)
            wait_scatter((ich - 1) % NBUF)

        if do_count:
            pltpu.sync_copy(cnt_v, cnt_hbm.at[s * N_CORES + c])
        plsc.subcore_barrier()

        pltpu.sync_copy(acc.at[pl.ds(base, rows_per_sub)],
                        out_hbm.at[c, pl.ds(base, rows_per_sub)])

    return seg_sum


_ROW_BLK = 2000


def _mm2_kernel(x_ref, wa_ref, wb_ref, a_ref, b_ref):
    xb = x_ref[...]
    a_ref[...] = jnp.dot(xb, wa_ref[...], preferred_element_type=jnp.float32)
    b_ref[...] = jnp.dot(xb, wb_ref[...], preferred_element_type=jnp.float32)


def _mm2(x, wa_t, wb_t):
    n, d = x.shape
    grid = n // _ROW_BLK
    w_spec = pl.BlockSpec((d, wa_t.shape[1]), lambda i: (0, 0))
    row_spec = pl.BlockSpec((_ROW_BLK, d), lambda i: (i, 0))
    out_spec = pl.BlockSpec((_ROW_BLK, wa_t.shape[1]), lambda i: (i, 0))
    return pl.pallas_call(
        _mm2_kernel,
        grid=(grid,),
        in_specs=[row_spec, w_spec, w_spec],
        out_specs=[out_spec, out_spec],
        out_shape=[
            jax.ShapeDtypeStruct((n, wa_t.shape[1]), jnp.float32),
            jax.ShapeDtypeStruct((n, wb_t.shape[1]), jnp.float32),
        ],
    )(x, wa_t, wb_t)


def _combine_mm2_kernel(aggp_ref, cntp_ref, r_ref, bl_ref, wa_ref, wb_ref,
                        a_ref, b_ref):
    agg = aggp_ref[0] + aggp_ref[1]
    cnt = jnp.sum(cntp_ref[...], axis=1)[:, None]
    mean = agg / jnp.maximum(cnt, 1.0)
    h = jnp.maximum(mean + bl_ref[...] + r_ref[...], 0.0)
    a_ref[...] = jnp.dot(h, wa_ref[...], preferred_element_type=jnp.float32)
    b_ref[...] = jnp.dot(h, wb_ref[...], preferred_element_type=jnp.float32)


def _combine_mm2(aggp, cntp, r, bl, wa_t, wb_t):
    n, d = r.shape
    grid = n // _ROW_BLK
    return pl.pallas_call(
        _combine_mm2_kernel,
        grid=(grid,),
        in_specs=[
            pl.BlockSpec((N_CORES, _ROW_BLK, d), lambda i: (0, i, 0)),
            pl.BlockSpec((_ROW_BLK, N_WORKERS), lambda i: (i, 0)),
            pl.BlockSpec((_ROW_BLK, d), lambda i: (i, 0)),
            pl.BlockSpec((d,), lambda i: (0,)),
            pl.BlockSpec((d, wa_t.shape[1]), lambda i: (0, 0)),
            pl.BlockSpec((d, wb_t.shape[1]), lambda i: (0, 0)),
        ],
        out_specs=[
            pl.BlockSpec((_ROW_BLK, wa_t.shape[1]), lambda i: (i, 0)),
            pl.BlockSpec((_ROW_BLK, wb_t.shape[1]), lambda i: (i, 0)),
        ],
        out_shape=[
            jax.ShapeDtypeStruct((n, wa_t.shape[1]), jnp.float32),
            jax.ShapeDtypeStruct((n, wb_t.shape[1]), jnp.float32),
        ],
    )(aggp, cntp, r, bl, wa_t, wb_t)


def _combine_out_kernel(aggp_ref, cntp_ref, r_ref, bl_ref, w_ref, blin_ref,
                        o_ref):
    agg = aggp_ref[0] + aggp_ref[1]
    cnt = jnp.sum(cntp_ref[...], axis=1)[:, None]
    mean = agg / jnp.maximum(cnt, 1.0)
    h = jnp.maximum(mean + bl_ref[...] + r_ref[...], 0.0)
    o_ref[...] = (
        jnp.dot(h, w_ref[...], preferred_element_type=jnp.float32)
        + blin_ref[...]
    )


def _combine_out(aggp, cntp, r, bl, w_t, blin):
    n, d = r.shape
    grid = n // _ROW_BLK
    return pl.pallas_call(
        _combine_out_kernel,
        grid=(grid,),
        in_specs=[
            pl.BlockSpec((N_CORES, _ROW_BLK, d), lambda i: (0, i, 0)),
            pl.BlockSpec((_ROW_BLK, N_WORKERS), lambda i: (i, 0)),
            pl.BlockSpec((_ROW_BLK, d), lambda i: (i, 0)),
            pl.BlockSpec((d,), lambda i: (0,)),
            pl.BlockSpec((d, w_t.shape[1]), lambda i: (0, 0)),
            pl.BlockSpec((w_t.shape[1],), lambda i: (0,)),
        ],
        out_specs=pl.BlockSpec((_ROW_BLK, w_t.shape[1]), lambda i: (i, 0)),
        out_shape=jax.ShapeDtypeStruct((n, w_t.shape[1]), jnp.float32),
    )(aggp, cntp, r, bl, w_t, blin)


def kernel(x, edge_index, Wl1, bl1, Wr1, Wl2, bl2, Wr2, Wlin, blin):
    n, d = x.shape
    e = edge_index.shape[1]

    nbt = -(-e // (N_SUBCORES * BATCH * IDX_CHUNK)) * IDX_CHUNK
    nb0 = min(nbt, max(IDX_CHUNK, round(0.75 * nbt / IDX_CHUNK) * IDX_CHUNK))
    nb1 = nbt - nb0
    e_pad = N_SUBCORES * nbt * BATCH
    n_pad = -(-(n + 1) // (N_SUBCORES * 8)) * (N_SUBCORES * 8)

    pad = e_pad - e
    src = jnp.concatenate(
        [edge_index[0], jnp.zeros((pad,), jnp.int32)]
    ).reshape(N_SUBCORES, nbt, BATCH)
    dst = jnp.concatenate(
        [edge_index[1], jnp.full((pad,), n, jnp.int32)]
    ).reshape(N_SUBCORES, nbt, BATCH)
    seg_sum_cnt = _make_seg_sum(n_pad, d, nb0, nb1, do_count=True)
    ich2 = 40 if (nb0 % 40 == 0 and nb1 % 40 == 0) else IDX_CHUNK
    seg_sum = _make_seg_sum(n_pad, d, nb0, nb1, ich=ich2)

    y1, r1 = _mm2(x, Wl1.T, Wr1.T)
    aggp1, cntp = seg_sum_cnt(y1, src, dst)
    cntp = cntp.T
    y2, r2 = _combine_mm2(aggp1, cntp, r1, bl1, Wl2.T, Wr2.T)
    aggp2 = seg_sum(y2, src, dst)
    return _combine_out(aggp2, cntp, r2, bl2, Wlin.T, blin)
